# baseline (speedup 1.0000x reference)
import functools

import jax
import jax.numpy as jnp
from jax import lax
from jax.experimental import pallas as pl
from jax.experimental.pallas import tpu as pltpu
from jax.experimental.pallas import tpu_sc as plsc

_B = 16384
_D = 32
_GAMMA = 1.0
_NC = 2
_NS = 16
_NW = _NC * _NS
_C = _B // _NW
_G = 128
_NG = _C // _G
_CHUNK = 16
_NCHUNK = _C // _CHUNK


def _rsqrt_nr(x):
    xh = x * jnp.float32(0.5)
    i = plsc.bitcast(x, jnp.int32)
    i = jnp.int32(0x5F3759DF) - jnp.right_shift(i, jnp.int32(1))
    y = plsc.bitcast(i, jnp.float32)
    for _ in range(3):
        y = y * (jnp.float32(1.5) - xh * y * y)
    return y


def _transe_body(head, tail, rel, n_head, n_tail, n_rel, ent, rel_emb, out,
                 hix, tix, rix, nhix, ntix, nrix,
                 hrow, trow, rrow, nhrow, ntrow, nrrow,
                 outv, sem):
    wid = lax.axis_index("s") * _NC + lax.axis_index("c")
    base = wid * _C

    for j in range(_NG):
        sl = pl.ds(base + j * _G, _G)
        pltpu.sync_copy(head.at[sl], hix.at[j])
        pltpu.sync_copy(tail.at[sl], tix.at[j])
        pltpu.sync_copy(rel.at[sl], rix.at[j])
        pltpu.sync_copy(n_head.at[sl], nhix.at[j])
        pltpu.sync_copy(n_tail.at[sl], ntix.at[j])
        pltpu.sync_copy(n_rel.at[sl], nrix.at[j])

    handles = []
    for j in range(_NG):
        dsl = pl.ds(j * _G, _G)
        handles.append(pltpu.async_copy(ent.at[hix.at[j]], hrow.at[dsl], sem))
        handles.append(pltpu.async_copy(ent.at[tix.at[j]], trow.at[dsl], sem))
        handles.append(pltpu.async_copy(rel_emb.at[rix.at[j]], rrow.at[dsl], sem))
        handles.append(pltpu.async_copy(ent.at[nhix.at[j]], nhrow.at[dsl], sem))
        handles.append(pltpu.async_copy(ent.at[ntix.at[j]], ntrow.at[dsl], sem))
        handles.append(pltpu.async_copy(rel_emb.at[nrix.at[j]], nrrow.at[dsl], sem))
    for h in handles:
        h.wait()

    zero = jnp.zeros((_CHUNK,), jnp.float32)

    def chunk_body(c, carry):
        rows = c * _CHUNK + lax.iota(jnp.int32, _CHUNK)
        hh = tt = rr = hr = ht = rt = zero
        mhh = mtt = mrr = mhr = mht = mrt = zero
        for d in range(_D):
            col = jnp.full((_CHUNK,), d, jnp.int32)
            h = plsc.load_gather(hrow, [rows, col])
            t = plsc.load_gather(trow, [rows, col])
            r = plsc.load_gather(rrow, [rows, col])
            nh = plsc.load_gather(nhrow, [rows, col])
            nt = plsc.load_gather(ntrow, [rows, col])
            nr = plsc.load_gather(nrrow, [rows, col])
            hh = hh + h * h
            tt = tt + t * t
            rr = rr + r * r
            hr = hr + h * r
            ht = ht + h * t
            rt = rt + r * t
            mhh = mhh + nh * nh
            mtt = mtt + nt * nt
            mrr = mrr + nr * nr
            mhr = mhr + nh * nr
            mht = mht + nh * nt
            mrt = mrt + nr * nt

        two = jnp.float32(2.0)
        three = jnp.float32(3.0)
        eps = jnp.float32(1e-30)
        pos2 = three + two * (hr * _rsqrt_nr(hh * rr)
                              - ht * _rsqrt_nr(hh * tt)
                              - rt * _rsqrt_nr(rr * tt))
        neg2 = three + two * (mhr * _rsqrt_nr(mhh * mrr)
                              - mht * _rsqrt_nr(mhh * mtt)
                              - mrt * _rsqrt_nr(mrr * mtt))
        pos2 = jnp.maximum(pos2, eps)
        neg2 = jnp.maximum(neg2, eps)
        pos = pos2 * _rsqrt_nr(pos2)
        neg = neg2 * _rsqrt_nr(neg2)
        outv[pl.ds(c * _CHUNK, _CHUNK)] = jnp.float32(_GAMMA) + pos - neg
        return carry

    lax.fori_loop(0, _NCHUNK, chunk_body, 0)
    pltpu.sync_copy(outv, out.at[pl.ds(base, _C)])


_transe_sc = functools.partial(
    pl.kernel,
    mesh=plsc.VectorSubcoreMesh(core_axis_name="c", subcore_axis_name="s"),
    compiler_params=pltpu.CompilerParams(
        needs_layout_passes=False, use_tc_tiling_on_sc=False),
    out_type=jax.ShapeDtypeStruct((_B,), jnp.float32),
    scratch_types=[
        pltpu.VMEM((_NG, _G), jnp.int32),
        pltpu.VMEM((_NG, _G), jnp.int32),
        pltpu.VMEM((_NG, _G), jnp.int32),
        pltpu.VMEM((_NG, _G), jnp.int32),
        pltpu.VMEM((_NG, _G), jnp.int32),
        pltpu.VMEM((_NG, _G), jnp.int32),
        pltpu.VMEM((_C, _D), jnp.float32),
        pltpu.VMEM((_C, _D), jnp.float32),
        pltpu.VMEM((_C, _D), jnp.float32),
        pltpu.VMEM((_C, _D), jnp.float32),
        pltpu.VMEM((_C, _D), jnp.float32),
        pltpu.VMEM((_C, _D), jnp.float32),
        pltpu.VMEM((_C,), jnp.float32),
        pltpu.SemaphoreType.DMA,
    ],
)(_transe_body)


def kernel(head, tail, relation, n_head, n_tail, n_relation, entity_embed, relation_embed):
    return _transe_sc(
        head.astype(jnp.int32),
        tail.astype(jnp.int32),
        relation.astype(jnp.int32),
        n_head.astype(jnp.int32),
        n_tail.astype(jnp.int32),
        n_relation.astype(jnp.int32),
        entity_embed,
        relation_embed,
    )


# speedup vs baseline: 1.0294x; 1.0294x over previous
import functools

import jax
import jax.numpy as jnp
from jax import lax
from jax.experimental import pallas as pl
from jax.experimental.pallas import tpu as pltpu
from jax.experimental.pallas import tpu_sc as plsc

_B = 16384
_D = 32
_GAMMA = 1.0
_NC = 2
_NS = 16
_NW = _NC * _NS
_C = _B // _NW
_G = 64
_NG = _C // _G
_CHUNK = 16
_NK = _G // _CHUNK
_E4 = 1000000 // 4
_R4 = 1000 // 4


def _rsqrt_nr(x):
    xh = x * jnp.float32(0.5)
    i = plsc.bitcast(x, jnp.int32)
    i = jnp.int32(0x5F3759DF) - jnp.right_shift(i, jnp.int32(1))
    y = plsc.bitcast(i, jnp.float32)
    for _ in range(3):
        y = y * (jnp.float32(1.5) - xh * y * y)
    return y


def _transe_body(head, tail, rel, n_head, n_tail, n_rel, ent, rel_emb, out,
                 ix0, ix1, ix2, ix3, ix4, ix5,
                 gx0, gx1, gx2, gx3, gx4, gx5,
                 gr0, gr1, gr2, gr3, gr4, gr5,
                 outv, sem):
    wid = lax.axis_index("s") * _NC + lax.axis_index("c")
    base = wid * _C

    srcs = (head, tail, rel, n_head, n_tail, n_rel)
    tables = (ent, ent, rel_emb, ent, ent, rel_emb)
    ix = (ix0, ix1, ix2, ix3, ix4, ix5)
    gx = (gx0, gx1, gx2, gx3, gx4, gx5)
    gr = (gr0, gr1, gr2, gr3, gr4, gr5)

    zero = jnp.zeros((_CHUNK,), jnp.float32)
    iota = lax.iota(jnp.int32, _CHUNK)

    def group_body(j, carry):
        gbase = base + j * _G
        for t in range(6):
            pltpu.sync_copy(srcs[t].at[pl.ds(gbase, _G)], ix[t])
        for t in range(6):
            for k in range(_NK):
                sl = pl.ds(k * _CHUNK, _CHUNK)
                gx[t][sl] = jnp.right_shift(ix[t][sl], jnp.int32(2))
        handles = [
            pltpu.async_copy(tables[t].at[gx[t]], gr[t], sem)
            for t in range(6)
        ]
        for h in handles:
            h.wait()

        def chunk_body(k, carry3):
            rows = k * _CHUNK + iota
            sl = pl.ds(k * _CHUNK, _CHUNK)
            cb = [
                jnp.bitwise_and(ix[t][sl], jnp.int32(3)) * jnp.int32(_D)
                for t in range(6)
            ]
            hh = tt = rr = hr = ht = rt = zero
            mhh = mtt = mrr = mhr = mht = mrt = zero
            for s in range(_D):
                diag = jnp.bitwise_and(iota + jnp.int32(s), jnp.int32(_D - 1))
                h = plsc.load_gather(gr0, [rows, cb[0] + diag])
                t_ = plsc.load_gather(gr1, [rows, cb[1] + diag])
                r_ = plsc.load_gather(gr2, [rows, cb[2] + diag])
                nh = plsc.load_gather(gr3, [rows, cb[3] + diag])
                nt = plsc.load_gather(gr4, [rows, cb[4] + diag])
                nr = plsc.load_gather(gr5, [rows, cb[5] + diag])
                hh = hh + h * h
                tt = tt + t_ * t_
                rr = rr + r_ * r_
                hr = hr + h * r_
                ht = ht + h * t_
                rt = rt + r_ * t_
                mhh = mhh + nh * nh
                mtt = mtt + nt * nt
                mrr = mrr + nr * nr
                mhr = mhr + nh * nr
                mht = mht + nh * nt
                mrt = mrt + nr * nt
            two = jnp.float32(2.0)
            three = jnp.float32(3.0)
            eps = jnp.float32(1e-30)
            pos2 = three + two * (hr * _rsqrt_nr(hh * rr)
                                  - ht * _rsqrt_nr(hh * tt)
                                  - rt * _rsqrt_nr(rr * tt))
            neg2 = three + two * (mhr * _rsqrt_nr(mhh * mrr)
                                  - mht * _rsqrt_nr(mhh * mtt)
                                  - mrt * _rsqrt_nr(mrr * mtt))
            pos2 = jnp.maximum(pos2, eps)
            neg2 = jnp.maximum(neg2, eps)
            pos = pos2 * _rsqrt_nr(pos2)
            neg = neg2 * _rsqrt_nr(neg2)
            outv[pl.ds((j * _NK + k) * _CHUNK, _CHUNK)] = (
                jnp.float32(_GAMMA) + pos - neg)
            return carry3

        lax.fori_loop(0, _NK, chunk_body, 0)
        return carry

    lax.fori_loop(0, _NG, group_body, 0)
    pltpu.sync_copy(outv, out.at[pl.ds(base, _C)])


_transe_sc = functools.partial(
    pl.kernel,
    mesh=plsc.VectorSubcoreMesh(core_axis_name="c", subcore_axis_name="s"),
    compiler_params=pltpu.CompilerParams(
        needs_layout_passes=False, use_tc_tiling_on_sc=False),
    out_type=jax.ShapeDtypeStruct((_B,), jnp.float32),
    scratch_types=(
        [pltpu.VMEM((_G,), jnp.int32) for _ in range(6)]
        + [pltpu.VMEM((_G,), jnp.int32) for _ in range(6)]
        + [pltpu.VMEM((_G, 128), jnp.float32) for _ in range(6)]
        + [pltpu.VMEM((_C,), jnp.float32),
           pltpu.SemaphoreType.DMA]
    ),
)(_transe_body)


def kernel(head, tail, relation, n_head, n_tail, n_relation, entity_embed, relation_embed):
    return _transe_sc(
        head.astype(jnp.int32),
        tail.astype(jnp.int32),
        relation.astype(jnp.int32),
        n_head.astype(jnp.int32),
        n_tail.astype(jnp.int32),
        n_relation.astype(jnp.int32),
        entity_embed.reshape(_E4, 128),
        relation_embed.reshape(_R4, 128),
    )
